# baseline (speedup 1.0000x reference)
import functools

import jax
import jax.numpy as jnp
from jax import lax
from jax.experimental import pallas as pl

_NE = 64
_NF = 16
_NPF = 4
_TOPK = 8
_R = 8
_AD = 2 * _NPF * _R


def _dotg(a, b, dims, out_dtype=jnp.float32):
    return lax.dot_general(a, b, (dims, ((), ())),
                           preferred_element_type=out_dtype)


def _token_kernel(h_ref, wg_ref, sgw_ref, suw_ref, sdw_ref,
                  qa_ref, amix_ref,
                  fwx_ref, sc16_ref, tgs_ref, sh_ref):
    h = h_ref[...]
    logits = _dotg(h, wg_ref[...], ((1,), (1,)))
    s = jax.nn.softmax(logits, axis=-1)
    ii = lax.broadcasted_iota(jnp.int32, s.shape, 1)
    cur = s
    selw = jnp.zeros_like(s)
    for _ in range(_TOPK):
        m = jnp.max(cur, axis=-1, keepdims=True)
        cand = jnp.where(cur == m, ii, _NE)
        j = jnp.min(cand, axis=-1, keepdims=True)
        hit = ii == j
        selw = jnp.where(hit, s, selw)
        cur = jnp.where(hit, -1.0, cur)
    ri = lax.broadcasted_iota(jnp.int32, (_NE, _NE), 0)
    ci = lax.broadcasted_iota(jnp.int32, (_NE, _NE), 1)
    gblock = (ri // _NPF == ci // _NPF).astype(jnp.float32)
    e_ = jnp.where(selw > 0.0, jnp.exp(selw), 0.0)
    denom = _dotg(e_, gblock, ((1,), (0,)))
    fwx = e_ / jnp.maximum(denom, 1e-30)
    fwx_ref[...] = fwx
    gi16 = lax.broadcasted_iota(jnp.int32, (_NE, _NF), 0)
    ci16 = lax.broadcasted_iota(jnp.int32, (_NE, _NF), 1)
    g16 = (gi16 // _NPF == ci16).astype(jnp.float32)
    sc16_ref[...] = _dotg(selw, g16, ((1,), (0,)))
    hb = h.astype(jnp.bfloat16)
    fwx_b = fwx.astype(jnp.bfloat16)
    gi = lax.broadcasted_iota(jnp.int32, (_NPF, _AD), 0)
    li = lax.broadcasted_iota(jnp.int32, (_NPF, _AD), 1)
    rep = (gi == (li % (_NPF * _R)) // _R).astype(jnp.bfloat16)
    for e in range(_NF):
        t1 = _dotg(hb, qa_ref[e], ((1,), (1,)))
        fe = fwx_b[:, e * _NPF:(e + 1) * _NPF]
        t1 = t1 + _dotg(fe, amix_ref[e], ((1,), (0,)))
        scale = _dotg(fe, rep, ((1,), (0,)))
        tgs_ref[e, :, :] = (t1 * scale).astype(jnp.bfloat16)
    g = _dotg(hb, sgw_ref[...].astype(jnp.bfloat16), ((1,), (1,)))
    u = _dotg(hb, suw_ref[...].astype(jnp.bfloat16), ((1,), (1,)))
    gu = (jax.nn.silu(g) * u).astype(jnp.bfloat16)
    sh = _dotg(gu, sdw_ref[...].astype(jnp.bfloat16), ((1,), (1,)))
    sh_ref[...] = sh.astype(jnp.bfloat16)


def _moe_kernel(hb_ref, fwx_ref, sc16_ref, tgs_ref, shb_ref,
                gw_ref, mg_ref, gqb_ref,
                uw_ref, mu_ref, uqb_ref,
                dw_ref, dqa_ref, dqb_ref,
                y_ref, *, bt):
    e = pl.program_id(0)
    t = pl.program_id(1)
    rows = pl.ds(t * bt, bt)

    ri = lax.broadcasted_iota(jnp.int32, (_NE, _NPF), 0)
    ci = lax.broadcasted_iota(jnp.int32, (_NE, _NPF), 1)
    sel4 = (ri == ci + e * _NPF).astype(jnp.float32)
    fwx = _dotg(fwx_ref[rows, :], sel4, ((1,), (0,)))
    fwx_b = fwx.astype(jnp.bfloat16)
    r16 = lax.broadcasted_iota(jnp.int32, (_NF, 1), 0)
    sel1 = (r16 == e).astype(jnp.float32)
    scalar = _dotg(sc16_ref[rows, :], sel1, ((1,), (0,)))

    hb = hb_ref[rows, :]
    t1s = tgs_ref[0]

    g = (_dotg(hb, gw_ref[0], ((1,), (1,)))
         + _dotg(fwx_b, mg_ref[0], ((1,), (0,)))
         + _dotg(t1s, gqb_ref[0], ((1,), (0,))))
    u = (_dotg(hb, uw_ref[0], ((1,), (1,)))
         + _dotg(fwx_b, mu_ref[0], ((1,), (0,)))
         + _dotg(t1s, uqb_ref[0], ((1,), (0,))))
    gu = (jax.nn.silu(g) * u).astype(jnp.bfloat16)

    t1d = _dotg(gu, dqa_ref[0], ((1,), (1,)))
    gi = lax.broadcasted_iota(jnp.int32, (_NPF, _NPF * _R), 0)
    li = lax.broadcasted_iota(jnp.int32, (_NPF, _NPF * _R), 1)
    rep = (gi == li // _R).astype(jnp.float32)
    fw32 = _dotg(fwx, rep, ((1,), (0,)))
    t1ds = (t1d * fw32).astype(jnp.bfloat16)
    d = (_dotg(gu, dw_ref[0], ((1,), (1,)))
         + _dotg(t1ds, dqb_ref[0], ((1,), (0,))))
    contrib = scalar * d

    @pl.when(e == 0)
    def _():
        y_ref[rows, :] = shb_ref[rows, :].astype(jnp.float32) + contrib

    @pl.when(e > 0)
    def _():
        y_ref[rows, :] += contrib


@functools.partial(jax.jit, static_argnames=("interpret",))
def _run(hidden_states, W_gate, mup_W, gW, g_qa, g_qb, g_sc,
         uW, u_qa, u_qb, u_sc, dW, d_qa, d_qb, d_sc,
         sh_gW, sh_uW, sh_dW, interpret=False):
    orig_shape = hidden_states.shape
    H = orig_shape[-1]
    h = hidden_states.reshape(-1, H)
    T = h.shape[0]
    I = gW.shape[1]
    bf = jnp.bfloat16

    hb = h.astype(bf)
    mupT = mup_W.transpose(0, 2, 1)
    qa_gu = jnp.concatenate(
        [g_qa.reshape(_NF, _NPF * _R, H),
         u_qa.reshape(_NF, _NPF * _R, H)], axis=1)
    amix = jnp.einsum('enh,ekh->enk', mupT, qa_gu)
    mg = jnp.einsum('enh,eih->eni', mupT, gW).astype(bf)
    mu = jnp.einsum('enh,eih->eni', mupT, uW).astype(bf)
    gqb = (g_qb.transpose(0, 1, 3, 2).reshape(_NF, _NPF * _R, I)
           * g_sc[:, None, :])
    uqb = (u_qb.transpose(0, 1, 3, 2).reshape(_NF, _NPF * _R, I)
           * u_sc[:, None, :])
    z = jnp.zeros_like(gqb)
    gqb64 = jnp.concatenate([gqb, z], axis=1).astype(bf)
    uqb64 = jnp.concatenate([z, uqb], axis=1).astype(bf)
    dqa = d_qa.reshape(_NF, _NPF * _R, I).astype(bf)
    dqb = (d_qb.transpose(0, 1, 3, 2).reshape(_NF, _NPF * _R, H)
           * d_sc[:, None, :]).astype(bf)
    qa_gu_b = qa_gu.astype(bf)
    amix_b = amix.astype(bf)
    gWb = gW.astype(bf)
    uWb = uW.astype(bf)
    dWb = dW.astype(bf)

    bta = min(512, T)
    nta = T // bta
    res3 = lambda t: (0, 0)
    fwx, sc16, tgs, shb = pl.pallas_call(
        _token_kernel,
        grid=(nta,),
        in_specs=[
            pl.BlockSpec((bta, H), lambda t: (t, 0)),
            pl.BlockSpec((_NE, H), res3),
            pl.BlockSpec(sh_gW.shape, res3),
            pl.BlockSpec(sh_uW.shape, res3),
            pl.BlockSpec(sh_dW.shape, res3),
            pl.BlockSpec((_NF, _AD, H), lambda t: (0, 0, 0)),
            pl.BlockSpec((_NF, _NPF, _AD), lambda t: (0, 0, 0)),
        ],
        out_specs=[
            pl.BlockSpec((bta, _NE), lambda t: (t, 0)),
            pl.BlockSpec((bta, _NF), lambda t: (t, 0)),
            pl.BlockSpec((_NF, bta, _AD), lambda t: (0, t, 0)),
            pl.BlockSpec((bta, H), lambda t: (t, 0)),
        ],
        out_shape=[
            jax.ShapeDtypeStruct((T, _NE), jnp.float32),
            jax.ShapeDtypeStruct((T, _NF), jnp.float32),
            jax.ShapeDtypeStruct((_NF, T, _AD), bf),
            jax.ShapeDtypeStruct((T, H), bf),
        ],
        interpret=interpret,
    )(h, W_gate, sh_gW, sh_uW, sh_dW, qa_gu_b, amix_b)

    bt = min(512, T)
    nt = T // bt
    res = lambda e, t: (0, 0)
    per_e = lambda e, t: (e, 0, 0)
    y = pl.pallas_call(
        functools.partial(_moe_kernel, bt=bt),
        grid=(_NF, nt),
        in_specs=[
            pl.BlockSpec((T, H), res),
            pl.BlockSpec((T, _NE), res),
            pl.BlockSpec((T, _NF), res),
            pl.BlockSpec((1, bt, _AD), lambda e, t: (e, t, 0)),
            pl.BlockSpec((T, H), res),
            pl.BlockSpec((1, I, H), per_e),
            pl.BlockSpec((1, _NPF, I), per_e),
            pl.BlockSpec((1, _AD, I), per_e),
            pl.BlockSpec((1, I, H), per_e),
            pl.BlockSpec((1, _NPF, I), per_e),
            pl.BlockSpec((1, _AD, I), per_e),
            pl.BlockSpec((1, H, I), per_e),
            pl.BlockSpec((1, _NPF * _R, I), per_e),
            pl.BlockSpec((1, _NPF * _R, H), per_e),
        ],
        out_specs=pl.BlockSpec((T, H), res),
        out_shape=jax.ShapeDtypeStruct((T, H), jnp.float32),
        interpret=interpret,
    )(hb, fwx, sc16, tgs, shb,
      gWb, mg, gqb64,
      uWb, mu, uqb64,
      dWb, dqa, dqb)

    return y.reshape(orig_shape)


def kernel(hidden_states, W_gate, mup_W, gW, g_qa, g_qb, g_sc,
           uW, u_qa, u_qb, u_sc, dW, d_qa, d_qb, d_sc,
           sh_gW, sh_uW, sh_dW, inv_mapping):
    del inv_mapping
    return _run(hidden_states, W_gate, mup_W, gW, g_qa, g_qb, g_sc,
                uW, u_qa, u_qb, u_sc, dW, d_qa, d_qb, d_sc,
                sh_gW, sh_uW, sh_dW)


# speedup vs baseline: 1.0391x; 1.0391x over previous
import functools

import jax
import jax.numpy as jnp
from jax import lax
from jax.experimental import pallas as pl

_NE = 64
_NF = 16
_NPF = 4
_TOPK = 8
_R = 8


def _dotg(a, b, dims, out_dtype=jnp.float32):
    return lax.dot_general(a, b, (dims, ((), ())),
                           preferred_element_type=out_dtype)


def _gate_shared_kernel(h_ref, wg_ref, sgw_ref, suw_ref, sdw_ref,
                        gw_ref, uw_ref, dw_ref,
                        fw_ref, sh_ref, gwb_ref, uwb_ref, dwb_ref):
    h = h_ref[...]
    gwb_ref[...] = gw_ref[...].astype(jnp.bfloat16)
    uwb_ref[...] = uw_ref[...].astype(jnp.bfloat16)
    dwb_ref[...] = dw_ref[...].astype(jnp.bfloat16)
    logits = _dotg(h, wg_ref[...], ((1,), (1,)))
    s = jax.nn.softmax(logits, axis=-1)
    ii = lax.broadcasted_iota(jnp.int32, s.shape, 1)
    cur = s
    selw = jnp.zeros_like(s)
    for _ in range(_TOPK):
        m = jnp.max(cur, axis=-1, keepdims=True)
        cand = jnp.where(cur == m, ii, _NE)
        j = jnp.min(cand, axis=-1, keepdims=True)
        hit = ii == j
        selw = jnp.where(hit, s, selw)
        cur = jnp.where(hit, -1.0, cur)
    fw_ref[...] = selw
    hb = h.astype(jnp.bfloat16)
    g = _dotg(hb, sgw_ref[...].astype(jnp.bfloat16), ((1,), (1,)))
    u = _dotg(hb, suw_ref[...].astype(jnp.bfloat16), ((1,), (1,)))
    gu = (jax.nn.silu(g) * u).astype(jnp.bfloat16)
    sh = _dotg(gu, sdw_ref[...].astype(jnp.bfloat16), ((1,), (1,)))
    sh_ref[...] = sh.astype(jnp.bfloat16)


def _moe_kernel(hb_ref, fw_ref, shb_ref,
                gw_ref, gqa_ref, gqb_ref, gsc_ref,
                uw_ref, uqa_ref, uqb_ref, usc_ref,
                dw_ref, dqa_ref, dqb_ref, dsc_ref,
                mup_ref, y_ref, *, bt):
    e = pl.program_id(0)
    t = pl.program_id(1)
    rows = pl.ds(t * bt, bt)

    fw_all = fw_ref[rows, :]
    ri = lax.broadcasted_iota(jnp.int32, (_NE, _NPF), 0)
    ci = lax.broadcasted_iota(jnp.int32, (_NE, _NPF), 1)
    sel = (ri == ci + e * _NPF).astype(jnp.float32)
    fw_raw = _dotg(fw_all, sel, ((1,), (0,)))
    scalar = jnp.sum(fw_raw, axis=-1, keepdims=True)
    fwx = jax.nn.softmax(
        jnp.where(fw_raw == 0.0, -1e9, fw_raw), axis=-1)
    gi = lax.broadcasted_iota(jnp.int32, (_NPF, _NPF * _R), 0)
    li = lax.broadcasted_iota(jnp.int32, (_NPF, _NPF * _R), 1)
    rep = (gi == li // _R).astype(jnp.float32)
    fw32 = _dotg(fwx, rep, ((1,), (0,)))

    x = hb_ref[rows, :].astype(jnp.float32) + _dotg(fwx, mup_ref[0],
                                                    ((1,), (0,)))
    xb = x.astype(jnp.bfloat16)

    def fused(inp_b, w_ref, qa_ref, qb_ref, sc_ref):
        main = _dotg(inp_b, w_ref[0], ((1,), (1,)))
        t1 = _dotg(inp_b, qa_ref[0], ((1,), (1,)))
        t1 = (t1 * fw32).astype(jnp.bfloat16)
        t2 = _dotg(t1, qb_ref[0], ((1,), (0,)))
        return main + sc_ref[0] * t2

    g = jax.nn.silu(fused(xb, gw_ref, gqa_ref, gqb_ref, gsc_ref))
    u = fused(xb, uw_ref, uqa_ref, uqb_ref, usc_ref)
    gu = (g * u).astype(jnp.bfloat16)
    d = fused(gu, dw_ref, dqa_ref, dqb_ref, dsc_ref)
    contrib = scalar * d

    @pl.when(e == 0)
    def _():
        y_ref[rows, :] = shb_ref[rows, :].astype(jnp.float32) + contrib

    @pl.when(e > 0)
    def _():
        y_ref[rows, :] += contrib


@functools.partial(jax.jit, static_argnames=("interpret",))
def _run(hidden_states, W_gate, mup_W, gW, g_qa, g_qb, g_sc,
         uW, u_qa, u_qb, u_sc, dW, d_qa, d_qb, d_sc,
         sh_gW, sh_uW, sh_dW, interpret=False):
    orig_shape = hidden_states.shape
    H = orig_shape[-1]
    h = hidden_states.reshape(-1, H)
    T = h.shape[0]
    I = gW.shape[1]
    bf = jnp.bfloat16

    hb = h.astype(bf)
    gqa = g_qa.reshape(_NF, _NPF * _R, H).astype(bf)
    uqa = u_qa.reshape(_NF, _NPF * _R, H).astype(bf)
    dqa = d_qa.reshape(_NF, _NPF * _R, I).astype(bf)
    gqb = g_qb.transpose(0, 1, 3, 2).reshape(_NF, _NPF * _R, I).astype(bf)
    uqb = u_qb.transpose(0, 1, 3, 2).reshape(_NF, _NPF * _R, I).astype(bf)
    dqb = d_qb.transpose(0, 1, 3, 2).reshape(_NF, _NPF * _R, H).astype(bf)
    mupT = mup_W.transpose(0, 2, 1)
    gsc = g_sc.reshape(_NF, 1, I)
    usc = u_sc.reshape(_NF, 1, I)
    dsc = d_sc.reshape(_NF, 1, H)

    nta = 4 * _NF
    bta = T // nta
    res3 = lambda t: (0, 0)
    half_iu = lambda t: (t // 4, t % 4, 0)
    fw, shb, gWb, uWb, dWb = pl.pallas_call(
        _gate_shared_kernel,
        grid=(nta,),
        in_specs=[
            pl.BlockSpec((bta, H), lambda t: (t, 0)),
            pl.BlockSpec((_NE, H), res3),
            pl.BlockSpec(sh_gW.shape, res3),
            pl.BlockSpec(sh_uW.shape, res3),
            pl.BlockSpec(sh_dW.shape, res3),
            pl.BlockSpec((1, I // 4, H), half_iu),
            pl.BlockSpec((1, I // 4, H), half_iu),
            pl.BlockSpec((1, H // 4, I), half_iu),
        ],
        out_specs=[
            pl.BlockSpec((bta, _NE), lambda t: (t, 0)),
            pl.BlockSpec((bta, H), lambda t: (t, 0)),
            pl.BlockSpec((1, I // 4, H), half_iu),
            pl.BlockSpec((1, I // 4, H), half_iu),
            pl.BlockSpec((1, H // 4, I), half_iu),
        ],
        out_shape=[
            jax.ShapeDtypeStruct((T, _NE), jnp.float32),
            jax.ShapeDtypeStruct((T, H), bf),
            jax.ShapeDtypeStruct((_NF, I, H), bf),
            jax.ShapeDtypeStruct((_NF, I, H), bf),
            jax.ShapeDtypeStruct((_NF, H, I), bf),
        ],
        interpret=interpret,
    )(h, W_gate, sh_gW, sh_uW, sh_dW, gW, uW, dW)

    bt = min(512, T)
    nt = T // bt
    res = lambda e, t: (0, 0)
    per_e = lambda e, t: (e, 0, 0)
    y = pl.pallas_call(
        functools.partial(_moe_kernel, bt=bt),
        grid=(_NF, nt),
        in_specs=[
            pl.BlockSpec((T, H), res),
            pl.BlockSpec((T, _NE), res),
            pl.BlockSpec((T, H), res),
            pl.BlockSpec((1, I, H), per_e),
            pl.BlockSpec((1, _NPF * _R, H), per_e),
            pl.BlockSpec((1, _NPF * _R, I), per_e),
            pl.BlockSpec((1, 1, I), per_e),
            pl.BlockSpec((1, I, H), per_e),
            pl.BlockSpec((1, _NPF * _R, H), per_e),
            pl.BlockSpec((1, _NPF * _R, I), per_e),
            pl.BlockSpec((1, 1, I), per_e),
            pl.BlockSpec((1, H, I), per_e),
            pl.BlockSpec((1, _NPF * _R, I), per_e),
            pl.BlockSpec((1, _NPF * _R, H), per_e),
            pl.BlockSpec((1, 1, H), per_e),
            pl.BlockSpec((1, _NPF, H), per_e),
        ],
        out_specs=pl.BlockSpec((T, H), res),
        out_shape=jax.ShapeDtypeStruct((T, H), jnp.float32),
        interpret=interpret,
    )(hb, fw, shb,
      gWb, gqa, gqb, gsc,
      uWb, uqa, uqb, usc,
      dWb, dqa, dqb, dsc,
      mupT)

    return y.reshape(orig_shape)


def kernel(hidden_states, W_gate, mup_W, gW, g_qa, g_qb, g_sc,
           uW, u_qa, u_qb, u_sc, dW, d_qa, d_qb, d_sc,
           sh_gW, sh_uW, sh_dW, inv_mapping):
    del inv_mapping
    return _run(hidden_states, W_gate, mup_W, gW, g_qa, g_qb, g_sc,
                uW, u_qa, u_qb, u_sc, dW, d_qa, d_qb, d_sc,
                sh_gW, sh_uW, sh_dW)


# speedup vs baseline: 1.0878x; 1.0469x over previous
import functools

import jax
import jax.numpy as jnp
from jax import lax
from jax.experimental import pallas as pl

_NE = 64
_NF = 16
_NPF = 4
_TOPK = 8
_R = 8


def _dotg(a, b, dims, out_dtype=jnp.float32):
    return lax.dot_general(a, b, (dims, ((), ())),
                           preferred_element_type=out_dtype)


def _gate_shared_kernel(h_ref, wg_ref, sgw_ref, suw_ref, sdw_ref,
                        fw_ref, sh_ref):
    h = h_ref[...]
    logits = _dotg(h, wg_ref[...], ((1,), (1,)))
    s = jax.nn.softmax(logits, axis=-1)
    ii = lax.broadcasted_iota(jnp.int32, s.shape, 1)
    cur = s
    selw = jnp.zeros_like(s)
    for _ in range(_TOPK):
        m = jnp.max(cur, axis=-1, keepdims=True)
        cand = jnp.where(cur == m, ii, _NE)
        j = jnp.min(cand, axis=-1, keepdims=True)
        hit = ii == j
        selw = jnp.where(hit, s, selw)
        cur = jnp.where(hit, -1.0, cur)
    fw_ref[...] = selw
    hb = h.astype(jnp.bfloat16)
    g = _dotg(hb, sgw_ref[...].astype(jnp.bfloat16), ((1,), (1,)))
    u = _dotg(hb, suw_ref[...].astype(jnp.bfloat16), ((1,), (1,)))
    gu = (jax.nn.silu(g) * u).astype(jnp.bfloat16)
    sh = _dotg(gu, sdw_ref[...].astype(jnp.bfloat16), ((1,), (1,)))
    sh_ref[...] = sh.astype(jnp.bfloat16)


def _moe_kernel(hb_ref, fw_ref, shb_ref,
                gw_ref, guqa_ref, gqb_ref,
                uw_ref, uqb_ref,
                dw_ref, dqa_ref, dqb_ref,
                mup_ref, y_ref, *, bt):
    e = pl.program_id(0)
    t = pl.program_id(1)
    rows = pl.ds(t * bt, bt)

    fw_all = fw_ref[rows, :]
    ri = lax.broadcasted_iota(jnp.int32, (_NE, _NPF), 0)
    ci = lax.broadcasted_iota(jnp.int32, (_NE, _NPF), 1)
    sel = (ri == ci + e * _NPF).astype(jnp.float32)
    fw_raw = _dotg(fw_all, sel, ((1,), (0,)))
    scalar = jnp.sum(fw_raw, axis=-1, keepdims=True)
    fwx = jax.nn.softmax(
        jnp.where(fw_raw == 0.0, -1e9, fw_raw), axis=-1)
    gi = lax.broadcasted_iota(jnp.int32, (_NPF, _NPF * _R), 0)
    li = lax.broadcasted_iota(jnp.int32, (_NPF, _NPF * _R), 1)
    rep = (gi == li // _R).astype(jnp.float32)
    fw32 = _dotg(fwx, rep, ((1,), (0,)))
    gi2 = lax.broadcasted_iota(jnp.int32, (_NPF, 2 * _NPF * _R), 0)
    li2 = lax.broadcasted_iota(jnp.int32, (_NPF, 2 * _NPF * _R), 1)
    rep2 = (gi2 == (li2 % (_NPF * _R)) // _R).astype(jnp.float32)
    fw64 = _dotg(fwx, rep2, ((1,), (0,)))

    x = hb_ref[rows, :].astype(jnp.float32) + _dotg(fwx, mup_ref[0],
                                                    ((1,), (0,)))
    xb = x.astype(jnp.bfloat16)

    t1gu = _dotg(xb, guqa_ref[0], ((1,), (1,)))
    t1gu_b = (t1gu * fw64).astype(jnp.bfloat16)
    g = jax.nn.silu(_dotg(xb, gw_ref[0], ((1,), (1,)))
                    + _dotg(t1gu_b, gqb_ref[0], ((1,), (0,))))
    u = (_dotg(xb, uw_ref[0], ((1,), (1,)))
         + _dotg(t1gu_b, uqb_ref[0], ((1,), (0,))))
    gu = (g * u).astype(jnp.bfloat16)
    t1d = _dotg(gu, dqa_ref[0], ((1,), (1,)))
    t1d_b = (t1d * fw32).astype(jnp.bfloat16)
    d = (_dotg(gu, dw_ref[0], ((1,), (1,)))
         + _dotg(t1d_b, dqb_ref[0], ((1,), (0,))))
    contrib = scalar * d

    @pl.when(e == 0)
    def _():
        y_ref[rows, :] = shb_ref[rows, :].astype(jnp.float32) + contrib

    @pl.when(e > 0)
    def _():
        y_ref[rows, :] += contrib


@functools.partial(jax.jit, static_argnames=("interpret",))
def _run(hidden_states, W_gate, mup_W, gW, g_qa, g_qb, g_sc,
         uW, u_qa, u_qb, u_sc, dW, d_qa, d_qb, d_sc,
         sh_gW, sh_uW, sh_dW, interpret=False):
    orig_shape = hidden_states.shape
    H = orig_shape[-1]
    h = hidden_states.reshape(-1, H)
    T = h.shape[0]
    I = gW.shape[1]
    bf = jnp.bfloat16

    hb = h.astype(bf)
    guqa = jnp.concatenate(
        [g_qa.reshape(_NF, _NPF * _R, H),
         u_qa.reshape(_NF, _NPF * _R, H)], axis=1).astype(bf)
    dqa = d_qa.reshape(_NF, _NPF * _R, I).astype(bf)
    gqb = (g_qb.transpose(0, 1, 3, 2).reshape(_NF, _NPF * _R, I)
           * g_sc[:, None, :])
    uqb = (u_qb.transpose(0, 1, 3, 2).reshape(_NF, _NPF * _R, I)
           * u_sc[:, None, :])
    z = jnp.zeros_like(gqb)
    gqb64 = jnp.concatenate([gqb, z], axis=1).astype(bf)
    uqb64 = jnp.concatenate([z, uqb], axis=1).astype(bf)
    dqb = (d_qb.transpose(0, 1, 3, 2).reshape(_NF, _NPF * _R, H)
           * d_sc[:, None, :]).astype(bf)
    mupT = mup_W.transpose(0, 2, 1)

    bta = min(512, T)
    nta = T // bta
    res3 = lambda t: (0, 0)
    fw, shb = pl.pallas_call(
        _gate_shared_kernel,
        grid=(nta,),
        in_specs=[
            pl.BlockSpec((bta, H), lambda t: (t, 0)),
            pl.BlockSpec((_NE, H), res3),
            pl.BlockSpec(sh_gW.shape, res3),
            pl.BlockSpec(sh_uW.shape, res3),
            pl.BlockSpec(sh_dW.shape, res3),
        ],
        out_specs=[
            pl.BlockSpec((bta, _NE), lambda t: (t, 0)),
            pl.BlockSpec((bta, H), lambda t: (t, 0)),
        ],
        out_shape=[
            jax.ShapeDtypeStruct((T, _NE), jnp.float32),
            jax.ShapeDtypeStruct((T, H), bf),
        ],
        interpret=interpret,
    )(h, W_gate, sh_gW, sh_uW, sh_dW)
    gWb = gW.astype(bf)
    uWb = uW.astype(bf)
    dWb = dW.astype(bf)

    bt = min(512, T)
    nt = T // bt
    res = lambda e, t: (0, 0)
    per_e = lambda e, t: (e, 0, 0)
    y = pl.pallas_call(
        functools.partial(_moe_kernel, bt=bt),
        grid=(_NF, nt),
        in_specs=[
            pl.BlockSpec((T, H), res),
            pl.BlockSpec((T, _NE), res),
            pl.BlockSpec((T, H), res),
            pl.BlockSpec((1, I, H), per_e),
            pl.BlockSpec((1, 2 * _NPF * _R, H), per_e),
            pl.BlockSpec((1, 2 * _NPF * _R, I), per_e),
            pl.BlockSpec((1, I, H), per_e),
            pl.BlockSpec((1, 2 * _NPF * _R, I), per_e),
            pl.BlockSpec((1, H, I), per_e),
            pl.BlockSpec((1, _NPF * _R, I), per_e),
            pl.BlockSpec((1, _NPF * _R, H), per_e),
            pl.BlockSpec((1, _NPF, H), per_e),
        ],
        out_specs=pl.BlockSpec((T, H), res),
        out_shape=jax.ShapeDtypeStruct((T, H), jnp.float32),
        interpret=interpret,
    )(hb, fw, shb,
      gWb, guqa, gqb64,
      uWb, uqb64,
      dWb, dqa, dqb,
      mupT)

    return y.reshape(orig_shape)


def kernel(hidden_states, W_gate, mup_W, gW, g_qa, g_qb, g_sc,
           uW, u_qa, u_qb, u_sc, dW, d_qa, d_qb, d_sc,
           sh_gW, sh_uW, sh_dW, inv_mapping):
    del inv_mapping
    return _run(hidden_states, W_gate, mup_W, gW, g_qa, g_qb, g_sc,
                uW, u_qa, u_qb, u_sc, dW, d_qa, d_qb, d_sc,
                sh_gW, sh_uW, sh_dW)


# speedup vs baseline: 1.2617x; 1.1598x over previous
import functools

import jax
import jax.numpy as jnp
from jax import lax
from jax.experimental import pallas as pl

_NE = 64
_NF = 16
_NPF = 4
_TOPK = 8
_R = 8


def _dotg(a, b, dims, out_dtype=jnp.float32):
    return lax.dot_general(a, b, (dims, ((), ())),
                           preferred_element_type=out_dtype)


def _gate_shared_kernel(h_ref, wg_ref, sgw_ref, suw_ref, sdw_ref,
                        fw_ref, sh_ref):
    h = h_ref[...]
    logits = _dotg(h, wg_ref[...], ((1,), (1,)))
    s = jax.nn.softmax(logits, axis=-1)
    ii = lax.broadcasted_iota(jnp.int32, s.shape, 1)
    cur = s
    selw = jnp.zeros_like(s)
    for _ in range(_TOPK):
        m = jnp.max(cur, axis=-1, keepdims=True)
        cand = jnp.where(cur == m, ii, _NE)
        j = jnp.min(cand, axis=-1, keepdims=True)
        hit = ii == j
        selw = jnp.where(hit, s, selw)
        cur = jnp.where(hit, -1.0, cur)
    fw_ref[...] = selw
    hb = h.astype(jnp.bfloat16)
    g = _dotg(hb, sgw_ref[...].astype(jnp.bfloat16), ((1,), (1,)))
    u = _dotg(hb, suw_ref[...].astype(jnp.bfloat16), ((1,), (1,)))
    gu = (jax.nn.silu(g) * u).astype(jnp.bfloat16)
    sh = _dotg(gu, sdw_ref[...].astype(jnp.bfloat16), ((1,), (1,)))
    sh_ref[...] = sh.astype(jnp.bfloat16)


def _moe_kernel(hb_ref, fw_ref, shb_ref,
                gw_ref, gqa_ref, gqb_ref, gsc_ref,
                uw_ref, uqa_ref, uqb_ref, usc_ref,
                dw_ref, dqa_ref, dqb_ref, dsc_ref,
                mup_ref, y_ref, *, bt):
    e = pl.program_id(0)
    t = pl.program_id(1)
    rows = pl.ds(t * bt, bt)

    fw_all = fw_ref[rows, :]
    ri = lax.broadcasted_iota(jnp.int32, (_NE, _NPF), 0)
    ci = lax.broadcasted_iota(jnp.int32, (_NE, _NPF), 1)
    sel = (ri == ci + e * _NPF).astype(jnp.float32)
    fw_raw = _dotg(fw_all, sel, ((1,), (0,)))
    scalar = jnp.sum(fw_raw, axis=-1, keepdims=True)
    fwx = jax.nn.softmax(
        jnp.where(fw_raw == 0.0, -1e9, fw_raw), axis=-1)
    gi = lax.broadcasted_iota(jnp.int32, (_NPF, _NPF * _R), 0)
    li = lax.broadcasted_iota(jnp.int32, (_NPF, _NPF * _R), 1)
    rep = (gi == li // _R).astype(jnp.float32)
    fw32 = _dotg(fwx, rep, ((1,), (0,)))

    x = hb_ref[rows, :].astype(jnp.float32) + _dotg(fwx, mup_ref[0],
                                                    ((1,), (0,)))
    xb = x.astype(jnp.bfloat16)

    def fused(inp_b, w_ref, qa_ref, qb_ref, sc_ref):
        main = _dotg(inp_b, w_ref[0], ((1,), (1,)))
        t1 = _dotg(inp_b, qa_ref[0], ((1,), (1,)))
        t1 = (t1 * fw32).astype(jnp.bfloat16)
        t2 = _dotg(t1, qb_ref[0], ((1,), (0,)))
        return main + sc_ref[0] * t2

    g = jax.nn.silu(fused(xb, gw_ref, gqa_ref, gqb_ref, gsc_ref))
    u = fused(xb, uw_ref, uqa_ref, uqb_ref, usc_ref)
    gu = (g * u).astype(jnp.bfloat16)
    d = fused(gu, dw_ref, dqa_ref, dqb_ref, dsc_ref)
    contrib = scalar * d

    @pl.when(e == 0)
    def _():
        y_ref[rows, :] = shb_ref[rows, :].astype(jnp.float32) + contrib

    @pl.when(e > 0)
    def _():
        y_ref[rows, :] += contrib


@functools.partial(jax.jit, static_argnames=("interpret",))
def _run(hidden_states, W_gate, mup_W, gW, g_qa, g_qb, g_sc,
         uW, u_qa, u_qb, u_sc, dW, d_qa, d_qb, d_sc,
         sh_gW, sh_uW, sh_dW, interpret=False):
    orig_shape = hidden_states.shape
    H = orig_shape[-1]
    h = hidden_states.reshape(-1, H)
    T = h.shape[0]
    I = gW.shape[1]
    bf = jnp.bfloat16

    hb = h.astype(bf)
    gqa = g_qa.reshape(_NF, _NPF * _R, H).astype(bf)
    uqa = u_qa.reshape(_NF, _NPF * _R, H).astype(bf)
    dqa = d_qa.reshape(_NF, _NPF * _R, I).astype(bf)
    gqb = g_qb.transpose(0, 1, 3, 2).reshape(_NF, _NPF * _R, I).astype(bf)
    uqb = u_qb.transpose(0, 1, 3, 2).reshape(_NF, _NPF * _R, I).astype(bf)
    dqb = d_qb.transpose(0, 1, 3, 2).reshape(_NF, _NPF * _R, H).astype(bf)
    mupT = mup_W.transpose(0, 2, 1)
    gsc = g_sc.reshape(_NF, 1, I)
    usc = u_sc.reshape(_NF, 1, I)
    dsc = d_sc.reshape(_NF, 1, H)
    gWb = gW.astype(bf)
    uWb = uW.astype(bf)
    dWb = dW.astype(bf)

    bta = min(512, T)
    nta = T // bta
    res3 = lambda t: (0, 0)
    fw, shb = pl.pallas_call(
        _gate_shared_kernel,
        grid=(nta,),
        in_specs=[
            pl.BlockSpec((bta, H), lambda t: (t, 0)),
            pl.BlockSpec((_NE, H), res3),
            pl.BlockSpec(sh_gW.shape, res3),
            pl.BlockSpec(sh_uW.shape, res3),
            pl.BlockSpec(sh_dW.shape, res3),
        ],
        out_specs=[
            pl.BlockSpec((bta, _NE), lambda t: (t, 0)),
            pl.BlockSpec((bta, H), lambda t: (t, 0)),
        ],
        out_shape=[
            jax.ShapeDtypeStruct((T, _NE), jnp.float32),
            jax.ShapeDtypeStruct((T, H), bf),
        ],
        interpret=interpret,
    )(h, W_gate, sh_gW, sh_uW, sh_dW)

    bt = min(512, T)
    nt = T // bt
    res = lambda e, t: (0, 0)
    per_e = lambda e, t: (e, 0, 0)
    y = pl.pallas_call(
        functools.partial(_moe_kernel, bt=bt),
        grid=(_NF, nt),
        in_specs=[
            pl.BlockSpec((T, H), res),
            pl.BlockSpec((T, _NE), res),
            pl.BlockSpec((T, H), res),
            pl.BlockSpec((1, I, H), per_e),
            pl.BlockSpec((1, _NPF * _R, H), per_e),
            pl.BlockSpec((1, _NPF * _R, I), per_e),
            pl.BlockSpec((1, 1, I), per_e),
            pl.BlockSpec((1, I, H), per_e),
            pl.BlockSpec((1, _NPF * _R, H), per_e),
            pl.BlockSpec((1, _NPF * _R, I), per_e),
            pl.BlockSpec((1, 1, I), per_e),
            pl.BlockSpec((1, H, I), per_e),
            pl.BlockSpec((1, _NPF * _R, I), per_e),
            pl.BlockSpec((1, _NPF * _R, H), per_e),
            pl.BlockSpec((1, 1, H), per_e),
            pl.BlockSpec((1, _NPF, H), per_e),
        ],
        out_specs=pl.BlockSpec((T, H), res),
        out_shape=jax.ShapeDtypeStruct((T, H), jnp.float32),
        interpret=interpret,
    )(hb, fw, shb,
      gWb, gqa, gqb, gsc,
      uWb, uqa, uqb, usc,
      dWb, dqa, dqb, dsc,
      mupT)

    return y.reshape(orig_shape)


def kernel(hidden_states, W_gate, mup_W, gW, g_qa, g_qb, g_sc,
           uW, u_qa, u_qb, u_sc, dW, d_qa, d_qb, d_sc,
           sh_gW, sh_uW, sh_dW, inv_mapping):
    del inv_mapping
    return _run(hidden_states, W_gate, mup_W, gW, g_qa, g_qb, g_sc,
                uW, u_qa, u_qb, u_sc, dW, d_qa, d_qb, d_sc,
                sh_gW, sh_uW, sh_dW)


# speedup vs baseline: 1.3220x; 1.0478x over previous
import functools

import jax
import jax.numpy as jnp
from jax import lax
from jax.experimental import pallas as pl

_NE = 64
_NF = 16
_NPF = 4
_TOPK = 8
_R = 8


def _dotg(a, b, dims, out_dtype=jnp.float32):
    return lax.dot_general(a, b, (dims, ((), ())),
                           preferred_element_type=out_dtype)


def _gate_shared_kernel(h_ref, wg_ref, sgw_ref, suw_ref, sdw_ref,
                        fw_ref, sh_ref):
    h = h_ref[...]
    logits = _dotg(h, wg_ref[...], ((1,), (1,)))
    s = jax.nn.softmax(logits, axis=-1)
    ii = lax.broadcasted_iota(jnp.int32, s.shape, 1)
    cur = s
    selw = jnp.zeros_like(s)
    for _ in range(_TOPK):
        m = jnp.max(cur, axis=-1, keepdims=True)
        cand = jnp.where(cur == m, ii, _NE)
        j = jnp.min(cand, axis=-1, keepdims=True)
        hit = ii == j
        selw = jnp.where(hit, s, selw)
        cur = jnp.where(hit, -1.0, cur)
    fw_ref[...] = selw
    hb = h.astype(jnp.bfloat16)
    g = _dotg(hb, sgw_ref[...].astype(jnp.bfloat16), ((1,), (1,)))
    u = _dotg(hb, suw_ref[...].astype(jnp.bfloat16), ((1,), (1,)))
    gu = (jax.nn.silu(g) * u).astype(jnp.bfloat16)
    sh = _dotg(gu, sdw_ref[...].astype(jnp.bfloat16), ((1,), (1,)))
    sh_ref[...] = sh.astype(jnp.bfloat16)


def _moe_kernel(hb_ref, fw_ref, shb_ref,
                gw_ref, gqa_ref, gqb_ref, gsc_ref,
                uw_ref, uqa_ref, uqb_ref, usc_ref,
                dw_ref, dqa_ref, dqb_ref, dsc_ref,
                mup_ref, y_ref, *, bt):
    e = pl.program_id(0)
    t = pl.program_id(1)
    rows = pl.ds(t * bt, bt)

    fw_all = fw_ref[rows, :]
    ri = lax.broadcasted_iota(jnp.int32, (_NE, _NPF), 0)
    ci = lax.broadcasted_iota(jnp.int32, (_NE, _NPF), 1)
    sel = (ri == ci + e * _NPF).astype(jnp.float32)
    fw_raw = _dotg(fw_all, sel, ((1,), (0,)))
    scalar = jnp.sum(fw_raw, axis=-1, keepdims=True)
    fwx = jax.nn.softmax(
        jnp.where(fw_raw == 0.0, -1e9, fw_raw), axis=-1)
    gi = lax.broadcasted_iota(jnp.int32, (_NPF, _NPF * _R), 0)
    li = lax.broadcasted_iota(jnp.int32, (_NPF, _NPF * _R), 1)
    rep = (gi == li // _R).astype(jnp.float32)
    fw32 = _dotg(fwx, rep, ((1,), (0,)))

    x = hb_ref[rows, :].astype(jnp.float32) + _dotg(fwx, mup_ref[0],
                                                    ((1,), (0,)))
    xb = x.astype(jnp.bfloat16)

    def fused(inp_b, w_ref, qa_ref, qb_ref, sc_ref):
        main = _dotg(inp_b, w_ref[0], ((1,), (1,)))
        t1 = _dotg(inp_b, qa_ref[0], ((1,), (1,)))
        t1 = (t1 * fw32).astype(jnp.bfloat16)
        t2 = _dotg(t1, qb_ref[0], ((1,), (0,)))
        return main + sc_ref[0] * t2

    g = jax.nn.silu(fused(xb, gw_ref, gqa_ref, gqb_ref, gsc_ref))
    u = fused(xb, uw_ref, uqa_ref, uqb_ref, usc_ref)
    gu = (g * u).astype(jnp.bfloat16)
    d = fused(gu, dw_ref, dqa_ref, dqb_ref, dsc_ref)
    contrib = scalar * d

    @pl.when(e == 0)
    def _():
        y_ref[rows, :] = shb_ref[rows, :].astype(jnp.float32) + contrib

    @pl.when(e > 0)
    def _():
        y_ref[rows, :] += contrib


@functools.partial(jax.jit, static_argnames=("interpret",))
def _run(hidden_states, W_gate, mup_W, gW, g_qa, g_qb, g_sc,
         uW, u_qa, u_qb, u_sc, dW, d_qa, d_qb, d_sc,
         sh_gW, sh_uW, sh_dW, interpret=False):
    orig_shape = hidden_states.shape
    H = orig_shape[-1]
    h = hidden_states.reshape(-1, H)
    T = h.shape[0]
    I = gW.shape[1]
    bf = jnp.bfloat16

    hb = h.astype(bf)
    gqa = g_qa.reshape(_NF, _NPF * _R, H).astype(bf)
    uqa = u_qa.reshape(_NF, _NPF * _R, H).astype(bf)
    dqa = d_qa.reshape(_NF, _NPF * _R, I).astype(bf)
    gqb = g_qb.transpose(0, 1, 3, 2).reshape(_NF, _NPF * _R, I).astype(bf)
    uqb = u_qb.transpose(0, 1, 3, 2).reshape(_NF, _NPF * _R, I).astype(bf)
    dqb = d_qb.transpose(0, 1, 3, 2).reshape(_NF, _NPF * _R, H).astype(bf)
    mupT = mup_W.transpose(0, 2, 1)
    gsc = g_sc.reshape(_NF, 1, I)
    usc = u_sc.reshape(_NF, 1, I)
    dsc = d_sc.reshape(_NF, 1, H)
    gWb = gW.astype(bf)
    uWb = uW.astype(bf)
    dWb = dW.astype(bf)

    bta = min(512, T)
    nta = T // bta
    res3 = lambda t: (0, 0)
    fw, shb = pl.pallas_call(
        _gate_shared_kernel,
        grid=(nta,),
        in_specs=[
            pl.BlockSpec((bta, H), lambda t: (t, 0)),
            pl.BlockSpec((_NE, H), res3),
            pl.BlockSpec(sh_gW.shape, res3),
            pl.BlockSpec(sh_uW.shape, res3),
            pl.BlockSpec(sh_dW.shape, res3),
        ],
        out_specs=[
            pl.BlockSpec((bta, _NE), lambda t: (t, 0)),
            pl.BlockSpec((bta, H), lambda t: (t, 0)),
        ],
        out_shape=[
            jax.ShapeDtypeStruct((T, _NE), jnp.float32),
            jax.ShapeDtypeStruct((T, H), bf),
        ],
        interpret=interpret,
    )(h, W_gate, sh_gW, sh_uW, sh_dW)

    bt = min(1024, T)
    nt = T // bt
    res = lambda e, t: (0, 0)
    per_e = lambda e, t: (e, 0, 0)
    y = pl.pallas_call(
        functools.partial(_moe_kernel, bt=bt),
        grid=(_NF, nt),
        in_specs=[
            pl.BlockSpec((T, H), res),
            pl.BlockSpec((T, _NE), res),
            pl.BlockSpec((T, H), res),
            pl.BlockSpec((1, I, H), per_e),
            pl.BlockSpec((1, _NPF * _R, H), per_e),
            pl.BlockSpec((1, _NPF * _R, I), per_e),
            pl.BlockSpec((1, 1, I), per_e),
            pl.BlockSpec((1, I, H), per_e),
            pl.BlockSpec((1, _NPF * _R, H), per_e),
            pl.BlockSpec((1, _NPF * _R, I), per_e),
            pl.BlockSpec((1, 1, I), per_e),
            pl.BlockSpec((1, H, I), per_e),
            pl.BlockSpec((1, _NPF * _R, I), per_e),
            pl.BlockSpec((1, _NPF * _R, H), per_e),
            pl.BlockSpec((1, 1, H), per_e),
            pl.BlockSpec((1, _NPF, H), per_e),
        ],
        out_specs=pl.BlockSpec((T, H), res),
        out_shape=jax.ShapeDtypeStruct((T, H), jnp.float32),
        interpret=interpret,
    )(hb, fw, shb,
      gWb, gqa, gqb, gsc,
      uWb, uqa, uqb, usc,
      dWb, dqa, dqb, dsc,
      mupT)

    return y.reshape(orig_shape)


def kernel(hidden_states, W_gate, mup_W, gW, g_qa, g_qb, g_sc,
           uW, u_qa, u_qb, u_sc, dW, d_qa, d_qb, d_sc,
           sh_gW, sh_uW, sh_dW, inv_mapping):
    del inv_mapping
    return _run(hidden_states, W_gate, mup_W, gW, g_qa, g_qb, g_sc,
                uW, u_qa, u_qb, u_sc, dW, d_qa, d_qb, d_sc,
                sh_gW, sh_uW, sh_dW)
